# Initial kernel scaffold; baseline (speedup 1.0000x reference)
import jax
import jax.numpy as jnp
from jax.experimental import pallas as pl


def kernel(pre, post, current_mask, current_W):
    raise NotImplementedError("write your pallas kernel here")


# speedup vs baseline: 12.8824x; 12.8824x over previous
import jax
import jax.numpy as jnp
from jax.experimental import pallas as pl

TARGET_DENSITY = 0.1
MAX_CHANGE = 4096
EXCL = 0x7FFFFFFF


def _keys(w, m8, a_col, b_row):
    ab = a_col * b_row
    s = jnp.abs(w) * ab
    kp = jax.lax.bitcast_convert_type(s, jnp.int32)
    kg = jnp.int32(EXCL) - jax.lax.bitcast_convert_type(ab, jnp.int32)
    act = m8 != 0
    kp = jnp.where(act, kp, jnp.int32(EXCL))
    kg = jnp.where(act, jnp.int32(EXCL), kg)
    return kp, kg


def _hist_body(w_ref, m_ref, a_ref, b_ref, pr_ref, hp_ref, hg_ref, cnt_ref):
    i = pl.program_id(0)

    @pl.when(i == 0)
    def _():
        hp_ref[...] = jnp.zeros_like(hp_ref)
        hg_ref[...] = jnp.zeros_like(hg_ref)
        cnt_ref[...] = jnp.zeros_like(cnt_ref)

    kp, kg = _keys(w_ref[...], m_ref[...], a_ref[...], b_ref[...])
    pmask_p = pr_ref[0, 0]
    pval_p = pr_ref[0, 1]
    pmask_g = pr_ref[0, 2]
    pval_g = pr_ref[0, 3]
    shift = pr_ref[0, 4]
    bmask = pr_ref[0, 5]

    mp = ((kp & pmask_p) == pval_p).astype(jnp.float32)
    mg = ((kg & pmask_g) == pval_g).astype(jnp.float32)
    binp = jnp.right_shift(kp, shift) & bmask
    bing = jnp.right_shift(kg, shift) & bmask

    br, n = kp.shape
    sr = 16
    width = (br * n) // sr
    binp2 = binp.reshape(sr, width)
    bing2 = bing.reshape(sr, width)
    mp2 = mp.reshape(sr, width)
    mg2 = mg.reshape(sr, width)
    iota16 = jax.lax.broadcasted_iota(jnp.int32, (16, 1), 0)

    accp = jnp.zeros((16, 16), jnp.float32)
    accg = jnp.zeros((16, 16), jnp.float32)
    dn = (((1,), (1,)), ((), ()))
    for r in range(sr):
        bp = binp2[r:r + 1, :]
        bg = bing2[r:r + 1, :]
        vp = mp2[r:r + 1, :]
        vg = mg2[r:r + 1, :]
        up = (iota16 == jnp.right_shift(bp, 4)).astype(jnp.float32) * vp
        lp = (iota16 == (bp & 15)).astype(jnp.float32)
        ug = (iota16 == jnp.right_shift(bg, 4)).astype(jnp.float32) * vg
        lg = (iota16 == (bg & 15)).astype(jnp.float32)
        accp = accp + jax.lax.dot_general(up, lp, dn, preferred_element_type=jnp.float32)
        accg = accg + jax.lax.dot_general(ug, lg, dn, preferred_element_type=jnp.float32)
    hp_ref[...] += accp
    hg_ref[...] += accg
    cnt_ref[...] += jnp.sum((m_ref[...] != 0).astype(jnp.float32)).reshape(1, 1)


def _minidx_body(w_ref, m_ref, a_ref, b_ref, pr_ref, mip_ref, mig_ref):
    i = pl.program_id(0)

    @pl.when(i == 0)
    def _():
        mip_ref[...] = jnp.full_like(mip_ref, jnp.int32(EXCL))
        mig_ref[...] = jnp.full_like(mig_ref, jnp.int32(EXCL))

    kp, kg = _keys(w_ref[...], m_ref[...], a_ref[...], b_ref[...])
    tp = pr_ref[0, 0]
    tg = pr_ref[0, 1]
    br, n = kp.shape
    r = jax.lax.broadcasted_iota(jnp.int32, (br, n), 0)
    c = jax.lax.broadcasted_iota(jnp.int32, (br, n), 1)
    flat = (i * br + r) * n + c
    mip_ref[...] = jnp.minimum(mip_ref[...], jnp.min(jnp.where(kp == tp, flat, jnp.int32(EXCL))).reshape(1, 1))
    mig_ref[...] = jnp.minimum(mig_ref[...], jnp.min(jnp.where(kg == tg, flat, jnp.int32(EXCL))).reshape(1, 1))


def _write_body(w_ref, m_ref, a_ref, b_ref, pr_ref, d_ref):
    i = pl.program_id(0)
    kp, kg = _keys(w_ref[...], m_ref[...], a_ref[...], b_ref[...])
    tp = pr_ref[0, 0]
    icp = pr_ref[0, 1]
    tg = pr_ref[0, 2]
    icg = pr_ref[0, 3]
    br, n = kp.shape
    r = jax.lax.broadcasted_iota(jnp.int32, (br, n), 0)
    c = jax.lax.broadcasted_iota(jnp.int32, (br, n), 1)
    flat = (i * br + r) * n + c
    selp = (kp < tp) | ((kp == tp) & (flat <= icp))
    selg = (kg < tg) | ((kg == tg) & (flat <= icg))
    d_ref[...] = (selg.astype(jnp.int32) - selp.astype(jnp.int32)).astype(jnp.int8)


def _means_body(pre_ref, post_ref, p_ref, q_ref, *, nb, rows):
    i = pl.program_id(0)

    @pl.when(i == 0)
    def _():
        p_ref[...] = jnp.zeros_like(p_ref)
        q_ref[...] = jnp.zeros_like(q_ref)

    p_ref[...] += jnp.sum(pre_ref[...], axis=0, keepdims=True)
    q_ref[...] += jnp.sum(post_ref[...], axis=0, keepdims=True)

    @pl.when(i == nb - 1)
    def _():
        p_ref[...] = p_ref[...] * (1.0 / rows)
        q_ref[...] = q_ref[...] * (1.0 / rows)


def _level(hist, r, pval, shift, bmask):
    h = hist.reshape(256)
    cum = jnp.cumsum(h)
    b = jnp.argmax(cum >= r).astype(jnp.int32)
    below = cum[b] - h[b]
    r2 = r - below
    pval2 = pval | (b << shift)
    return r2, pval2, h[b]


def kernel(pre, post, current_mask, current_W):
    n = current_W.shape[0]
    rows = pre.shape[0]
    total = n * n
    nb = max(1, n // 256)
    br = n // nb
    rb = rows // nb
    f32 = jnp.float32

    p = jnp.mean(pre.reshape(-1, pre.shape[-1]), axis=0)
    q = jnp.mean(post.reshape(-1, post.shape[-1]), axis=0)

    a = jnp.abs(q).reshape(n, 1)
    b = jnp.abs(p).reshape(1, n)
    m8 = current_mask.astype(jnp.int8)

    big_specs = [
        pl.BlockSpec((br, n), lambda i: (i, 0)),
        pl.BlockSpec((br, n), lambda i: (i, 0)),
        pl.BlockSpec((br, 1), lambda i: (i, 0)),
        pl.BlockSpec((1, n), lambda i: (0, 0)),
        pl.BlockSpec((1, 8), lambda i: (0, 0)),
    ]

    def hist_call(params):
        return pl.pallas_call(
            _hist_body,
            grid=(nb,),
            in_specs=big_specs,
            out_specs=[
                pl.BlockSpec((16, 16), lambda i: (0, 0)),
                pl.BlockSpec((16, 16), lambda i: (0, 0)),
                pl.BlockSpec((1, 1), lambda i: (0, 0)),
            ],
            out_shape=[
                jax.ShapeDtypeStruct((16, 16), f32),
                jax.ShapeDtypeStruct((16, 16), f32),
                jax.ShapeDtypeStruct((1, 1), f32),
            ],
        )(current_W, m8, a, b, params)

    shifts = (23, 15, 7, 0)
    bmasks = (0xFF, 0xFF, 0xFF, 0x7F)

    pmask_p = jnp.int32(0)
    pval_p = jnp.int32(0)
    pmask_g = jnp.int32(0)
    pval_g = jnp.int32(0)
    r_p = None
    r_g = None
    eq_p = eq_g = None
    active_n = None

    for lvl in range(4):
        sh = jnp.int32(shifts[lvl])
        bm = jnp.int32(bmasks[lvl])
        params = jnp.stack(
            [pmask_p, pval_p, pmask_g, pval_g, sh, bm, jnp.int32(0), jnp.int32(0)]
        ).reshape(1, 8)
        hp, hg, cnt = hist_call(params)
        if lvl == 0:
            active_n = cnt[0, 0]
            target_n = f32(int(TARGET_DENSITY * total))
            n_prune = jnp.clip(active_n - target_n, 0.0, float(MAX_CHANGE))
            n_grow = jnp.clip(target_n - active_n, 0.0, float(MAX_CHANGE))
            r_p = jnp.maximum(n_prune, 1.0)
            r_g = jnp.maximum(n_grow, 1.0)
        r_p, pval_p, eq_p = _level(hp, r_p, pval_p, shifts[lvl], bmasks[lvl])
        r_g, pval_g, eq_g = _level(hg, r_g, pval_g, shifts[lvl], bmasks[lvl])
        pmask_p = pmask_p | (jnp.int32(bmasks[lvl]) << shifts[lvl])
        pmask_g = pmask_g | (jnp.int32(bmasks[lvl]) << shifts[lvl])

    tparams = jnp.stack(
        [pval_p, pval_g, jnp.int32(0), jnp.int32(0), jnp.int32(0), jnp.int32(0),
         jnp.int32(0), jnp.int32(0)]
    ).reshape(1, 8)
    mip, mig = pl.pallas_call(
        _minidx_body,
        grid=(nb,),
        in_specs=big_specs,
        out_specs=[
            pl.BlockSpec((1, 1), lambda i: (0, 0)),
            pl.BlockSpec((1, 1), lambda i: (0, 0)),
        ],
        out_shape=[
            jax.ShapeDtypeStruct((1, 1), jnp.int32),
            jax.ShapeDtypeStruct((1, 1), jnp.int32),
        ],
    )(current_W, m8, a, b, tparams)

    def cuts(n_sel, r_fin, eq, pval, mi):
        none = n_sel < 0.5
        icut = jnp.where(r_fin >= eq, jnp.int32(total), mi)
        tv = jnp.where(none, jnp.int32(-1), pval)
        ic = jnp.where(none, jnp.int32(-1), icut)
        return tv, ic

    tp, icp = cuts(n_prune, r_p, eq_p, pval_p, mip[0, 0])
    tg, icg = cuts(n_grow, r_g, eq_g, pval_g, mig[0, 0])

    wparams = jnp.stack(
        [tp, icp, tg, icg, jnp.int32(0), jnp.int32(0), jnp.int32(0), jnp.int32(0)]
    ).reshape(1, 8)
    delta = pl.pallas_call(
        _write_body,
        grid=(nb,),
        in_specs=big_specs,
        out_specs=pl.BlockSpec((br, n), lambda i: (i, 0)),
        out_shape=jax.ShapeDtypeStruct((n, n), jnp.int8),
    )(current_W, m8, a, b, wparams)
    return delta
